# Initial kernel scaffold; baseline (speedup 1.0000x reference)
import jax
import jax.numpy as jnp
from jax.experimental import pallas as pl


def kernel(x, edge_index, W0, b0, W1, b1):
    raise NotImplementedError("write your pallas kernel here")


# speedup vs baseline: 6.0362x; 6.0362x over previous
import functools

import jax
import jax.numpy as jnp
from jax import lax
from jax.experimental import pallas as pl
from jax.experimental.pallas import tpu as pltpu
from jax.experimental.pallas import tpu_sc as plsc

N = 10000
D = 128
N_PAD = 10240
NC = 2
NS = 16
NW = NC * NS
CHUNK = 128
ROWS_PER_TILE = N_PAD // NS
ZCHUNK = 128

_MESH = dict(core_axis_name="c", subcore_axis_name="s")


def _make_spmm(cpt):
    @functools.partial(
        pl.kernel,
        out_type=jax.ShapeDtypeStruct((NC, N_PAD, D), jnp.float32),
        mesh=plsc.VectorSubcoreMesh(**_MESH),
        scratch_types=[
            pltpu.VMEM((cpt, CHUNK), jnp.int32),
            pltpu.VMEM((cpt, CHUNK), jnp.int32),
            pltpu.VMEM((_ZSTEPS, ZCHUNK), jnp.int32),
            pltpu.VMEM((CHUNK, D), jnp.float32),
            pltpu.VMEM_SHARED((N_PAD, D), jnp.float32),
        ],
    )
    def spmm(t_hbm, src_hbm, dst_hbm, zeros_hbm, rows_hbm, out_hbm,
             src_v, dst_v, rowidx_v, rows_v, acc_sh):
        cid = lax.axis_index("c")
        sid = lax.axis_index("s")
        wid = sid * NC + cid
        pltpu.sync_copy(src_hbm.at[wid], src_v)
        pltpu.sync_copy(dst_hbm.at[wid], dst_v)
        pltpu.sync_copy(rows_hbm.at[sid], rowidx_v)
        pltpu.sync_copy(zeros_hbm, rows_v)
        for z in range(_ZSTEPS):
            pltpu.sync_copy(rows_v, acc_sh.at[rowidx_v.at[z]])
        plsc.subcore_barrier()

        def body(j, carry):
            pltpu.sync_copy(t_hbm.at[src_v.at[j]], rows_v)
            pltpu.sync_copy(rows_v, acc_sh.at[dst_v.at[j]], add=True)
            return carry

        lax.fori_loop(0, cpt, body, 0)
        plsc.subcore_barrier()
        for z in range(_ZSTEPS):
            pltpu.sync_copy(acc_sh.at[rowidx_v.at[z]], rows_v)
            pltpu.sync_copy(
                rows_v,
                out_hbm.at[cid, pl.ds(sid * ROWS_PER_TILE + z * ZCHUNK,
                                      ZCHUNK)])

    return spmm


_ZSTEPS = ROWS_PER_TILE // ZCHUNK


_BLK = 1024
_GRID = N_PAD // _BLK


def _row(w):
    return pl.BlockSpec((_BLK, w), lambda i: (i, 0))


_W_SPEC = pl.BlockSpec((D, D), lambda i: (0, 0))
_B_SPEC = pl.BlockSpec((1, D), lambda i: (0, 0))
_PART_SPEC = pl.BlockSpec((NC, _BLK, D), lambda i: (0, i, 0))


def _mm(a, b):
    return jnp.dot(a, b, preferred_element_type=jnp.float32,
                   precision=lax.Precision.HIGHEST)


def _start_body(x_ref, d_ref, w_ref, dis_ref, dis2_ref, t_ref, acc_ref):
    deg = d_ref[0] + d_ref[1]
    dis = jnp.where(deg > 0, lax.rsqrt(jnp.maximum(deg, 1e-12)), 0.0)
    dis_ref[...] = dis
    dis2_ref[...] = dis * dis
    x = x_ref[...]
    t_ref[...] = dis * x
    acc_ref[...] = _mm(x, w_ref[...])


def _hop_mid_body(p_ref, dis_ref, dis2_ref, w_ref, accin_ref,
                  accout_ref, t_ref):
    s = p_ref[0] + p_ref[1]
    xk = dis_ref[...] * s
    accout_ref[...] = accin_ref[...] + _mm(xk, w_ref[...])
    t_ref[...] = dis2_ref[...] * s


def _end0_body(p_ref, dis_ref, w_ref, b_ref, wn_ref, accin_ref,
               t_ref, accout_ref):
    s = p_ref[0] + p_ref[1]
    xk = dis_ref[...] * s
    h = jnp.maximum(accin_ref[...] + _mm(xk, w_ref[...]) + b_ref[...], 0.0)
    t_ref[...] = dis_ref[...] * h
    accout_ref[...] = _mm(h, wn_ref[...])


def _end1_body(p_ref, dis_ref, w_ref, b_ref, accin_ref, out_ref):
    s = p_ref[0] + p_ref[1]
    xk = dis_ref[...] * s
    out_ref[...] = jnp.maximum(accin_ref[...] + _mm(xk, w_ref[...])
                               + b_ref[...], 0.0)


def _fmat(n=1):
    s = jax.ShapeDtypeStruct((N_PAD, D), jnp.float32)
    return [s] * n if n > 1 else s


_COL = jax.ShapeDtypeStruct((N_PAD, 1), jnp.float32)

_start = pl.pallas_call(
    _start_body, grid=(_GRID,),
    in_specs=[_row(D), pl.BlockSpec((NC, _BLK, 1), lambda i: (0, i, 0)),
              _W_SPEC],
    out_specs=[_row(1), _row(1), _row(D), _row(D)],
    out_shape=[_COL, _COL, _fmat(), _fmat()],
)

_hop_mid = pl.pallas_call(
    _hop_mid_body, grid=(_GRID,),
    in_specs=[_PART_SPEC, _row(1), _row(1), _W_SPEC, _row(D)],
    out_specs=[_row(D), _row(D)],
    out_shape=_fmat(2),
)

_end0 = pl.pallas_call(
    _end0_body, grid=(_GRID,),
    in_specs=[_PART_SPEC, _row(1), _W_SPEC, _B_SPEC, _W_SPEC, _row(D)],
    out_specs=[_row(D), _row(D)],
    out_shape=_fmat(2),
)

_end1 = pl.pallas_call(
    _end1_body, grid=(_GRID,),
    in_specs=[_PART_SPEC, _row(1), _W_SPEC, _B_SPEC, _row(D)],
    out_specs=_row(D),
    out_shape=_fmat(),
)


def kernel(x, edge_index, W0, b0, W1, b1):
    e = edge_index.shape[1]
    e_pad = ((e + NW * CHUNK - 1) // (NW * CHUNK)) * (NW * CHUNK)
    cpt = e_pad // (NW * CHUNK)
    src = edge_index[0].astype(jnp.int32)
    dst = edge_index[1].astype(jnp.int32)
    if e_pad != e:
        fill = jnp.full((e_pad - e,), N_PAD - 1, jnp.int32)
        src = jnp.concatenate([src, fill])
        dst = jnp.concatenate([dst, fill])
    src_p = src.reshape(NW, cpt, CHUNK)
    dst_p = dst.reshape(NW, cpt, CHUNK)
    x_p = jnp.pad(x, ((0, N_PAD - x.shape[0]), (0, 0)))

    zeros_rows = jnp.zeros((ZCHUNK, D), jnp.float32)

    spmm = _make_spmm(cpt)

    rows_idx = jnp.arange(N_PAD, dtype=jnp.int32).reshape(NS, _ZSTEPS, ZCHUNK)

    ones_tab = jnp.ones((N_PAD, D), jnp.float32)
    degout = spmm(ones_tab, src_p, dst_p, zeros_rows, rows_idx)
    dcol = degout[:, :, :1]

    b0r = b0.reshape(1, D)
    b1r = b1.reshape(1, D)

    dis, dis2, t, acc = _start(x_p, dcol, W0[0])
    for k in (1, 2):
        part = spmm(t, src_p, dst_p, zeros_rows, rows_idx)
        acc, t = _hop_mid(part, dis, dis2, W0[k], acc)
    part = spmm(t, src_p, dst_p, zeros_rows, rows_idx)
    t, acc = _end0(part, dis, W0[3], b0r, W1[0], acc)
    for k in (1, 2):
        part = spmm(t, src_p, dst_p, zeros_rows, rows_idx)
        acc, t = _hop_mid(part, dis, dis2, W1[k], acc)
    part = spmm(t, src_p, dst_p, zeros_rows, rows_idx)
    h = _end1(part, dis, W1[3], b1r, acc)
    return h[:N]
